# baseline (speedup 1.0000x reference)
import functools

import jax
import jax.numpy as jnp
from jax import lax
from jax.experimental import pallas as pl
from jax.experimental.pallas import tpu as pltpu
from jax.experimental.pallas import tpu_sc as plsc

_N_NODES = 10000
_N_EDGES = 320000
_OUT = 128
_NUM_LAYERS = 3

_NC, _NS = 2, 16
_NW = _NC * _NS
_EPW = _N_EDGES // _NW
_CH = 80
_NCH = _EPW // _CH
_DG = 25
_NG = _NCH // _DG
_NPAD = 10240
_RPS = _NPAD // _NS
_NWC = _RPS // _CH


def _seg_body(m_hbm, src_hbm, dst_hbm, out_hbm,
              srcs_v, dstg_v, rows_a, rows_b, acc_sh, sem_a, sem_b, sem_d):
    cid = lax.axis_index("c")
    sid = lax.axis_index("s")
    wid = cid * _NS + sid

    pltpu.make_async_copy(src_hbm.at[wid], srcs_v, sem_a).start()

    def zrow(i, carry):
        for j in range(_OUT // 16):
            rows_a[i, pl.ds(j * 16, 16)] = jnp.zeros((16,), jnp.float32)
        return carry
    lax.fori_loop(0, _CH, zrow, 0)

    def zcp(z, carry):
        pltpu.make_async_copy(
            rows_a, acc_sh.at[pl.ds(sid * _RPS + z * _CH, _CH)],
            sem_b).start()
        return carry
    lax.fori_loop(0, _NWC, zcp, 0)

    def zwait(z, carry):
        pltpu.make_async_copy(
            rows_a, acc_sh.at[pl.ds(sid * _RPS + z * _CH, _CH)],
            sem_b).wait()
        return carry
    lax.fori_loop(0, _NWC, zwait, 0)
    pltpu.make_async_copy(src_hbm.at[wid], srcs_v, sem_a).wait()
    pltpu.sync_copy(dst_hbm.at[wid, 0], dstg_v.at[pl.ds(0, _DG)])
    pltpu.make_async_copy(dst_hbm.at[wid, 1], dstg_v.at[pl.ds(_DG, _DG)],
                          sem_d).start()
    plsc.subcore_barrier()

    def load_dst_group(c):
        g = c // _DG

        @pl.when(jnp.logical_and(c % _DG == 0, g > 0))
        def _():
            pltpu.make_async_copy(
                dst_hbm.at[wid, g],
                dstg_v.at[pl.ds((g % 2) * _DG, _DG)], sem_d).wait()

            @pl.when(g + 1 < _NG)
            def _():
                pltpu.make_async_copy(
                    dst_hbm.at[wid, g + 1],
                    dstg_v.at[pl.ds(((g + 1) % 2) * _DG, _DG)],
                    sem_d).start()

    def gather_start(c, buf, sem):
        pltpu.make_async_copy(m_hbm.at[srcs_v.at[c]], buf, sem).start()

    def gather_wait(c, buf, sem):
        pltpu.make_async_copy(m_hbm.at[srcs_v.at[c]], buf, sem).wait()

    def scatter(c, buf):
        row = (c // _DG % 2) * _DG + c % _DG
        pltpu.sync_copy(buf, acc_sh.at[dstg_v.at[row]], add=True)

    gather_start(0, rows_a, sem_a)

    def pair(kk, carry):
        c0 = kk * 2
        gather_start(c0 + 1, rows_b, sem_b)
        load_dst_group(c0)
        gather_wait(c0, rows_a, sem_a)
        scatter(c0, rows_a)
        gather_start(c0 + 2, rows_a, sem_a)
        load_dst_group(c0 + 1)
        gather_wait(c0 + 1, rows_b, sem_b)
        scatter(c0 + 1, rows_b)
        return carry
    lax.fori_loop(0, (_NCH - 3) // 2, pair, 0)

    gather_start(_NCH - 2, rows_b, sem_b)
    gather_wait(_NCH - 3, rows_a, sem_a)
    scatter(_NCH - 3, rows_a)
    gather_start(_NCH - 1, rows_a, sem_a)
    load_dst_group(_NCH - 1)
    gather_wait(_NCH - 2, rows_b, sem_b)
    scatter(_NCH - 2, rows_b)
    gather_wait(_NCH - 1, rows_a, sem_a)
    scatter(_NCH - 1, rows_a)
    plsc.subcore_barrier()

    def wout(z, carry):
        r0 = sid * _RPS + z * _CH
        pltpu.make_async_copy(acc_sh.at[pl.ds(r0, _CH)],
                              out_hbm.at[cid, pl.ds(r0, _CH)], sem_b).start()
        return carry
    lax.fori_loop(0, _NWC, wout, 0)

    def wwait(z, carry):
        r0 = sid * _RPS + z * _CH
        pltpu.make_async_copy(acc_sh.at[pl.ds(r0, _CH)],
                              out_hbm.at[cid, pl.ds(r0, _CH)], sem_b).wait()
        return carry
    lax.fori_loop(0, _NWC, wwait, 0)


_seg_sum = functools.partial(
    pl.kernel,
    out_type=jax.ShapeDtypeStruct((_NC, _NPAD, _OUT), jnp.float32),
    mesh=plsc.VectorSubcoreMesh(core_axis_name="c", subcore_axis_name="s"),
    scratch_types=[
        pltpu.VMEM((_NCH, _CH), jnp.int32),
        pltpu.VMEM((2 * _DG, _CH), jnp.int32),
        pltpu.VMEM((_CH, _OUT), jnp.float32),
        pltpu.VMEM((_CH, _OUT), jnp.float32),
        pltpu.VMEM_SHARED((_NPAD, _OUT), jnp.float32),
        pltpu.SemaphoreType.DMA,
        pltpu.SemaphoreType.DMA,
        pltpu.SemaphoreType.DMA,
    ],
)(_seg_body)


_R = 2000


def _mm_body(x_ref, w_ref, o_ref):
    o_ref[...] = jnp.dot(x_ref[...], w_ref[...],
                         preferred_element_type=jnp.float32)


def _matmul(x, w):
    return pl.pallas_call(
        _mm_body,
        grid=(_N_NODES // _R,),
        in_specs=[pl.BlockSpec((_R, _OUT), lambda i: (i, 0)),
                  pl.BlockSpec((_OUT, _OUT), lambda i: (0, 0))],
        out_specs=pl.BlockSpec((_R, _OUT), lambda i: (i, 0)),
        out_shape=jax.ShapeDtypeStruct((_N_NODES, _OUT), jnp.float32),
    )(x, w)


def _gru_math(parts_ref, h, wihT, whhT, bih, bhh):
    agg = parts_ref[0] + parts_ref[1]
    gi = jnp.dot(agg, wihT, preferred_element_type=jnp.float32) + bih
    gh = jnp.dot(h, whhT, preferred_element_type=jnp.float32) + bhh
    r = jax.nn.sigmoid(gi[:, :_OUT] + gh[:, :_OUT])
    z = jax.nn.sigmoid(gi[:, _OUT:2 * _OUT] + gh[:, _OUT:2 * _OUT])
    n = jnp.tanh(gi[:, 2 * _OUT:] + r * gh[:, 2 * _OUT:])
    return (1.0 - z) * n + z * h


def _gru_mid_body(parts_ref, h_ref, wihT_ref, whhT_ref, bih_ref,
                  bhh_ref, wn_ref, hnew_ref, mnext_ref):
    hnew = _gru_math(parts_ref, h_ref[...], wihT_ref[...],
                     whhT_ref[...], bih_ref[...], bhh_ref[...])
    hnew_ref[...] = hnew
    mnext_ref[...] = jnp.dot(hnew, wn_ref[...],
                             preferred_element_type=jnp.float32)


def _gru_last_body(parts_ref, h_ref, wihT_ref, whhT_ref, bih_ref,
                   bhh_ref, linT_ref, lb_ref, out_ref):
    hnew = _gru_math(parts_ref, h_ref[...], wihT_ref[...],
                     whhT_ref[...], bih_ref[...], bhh_ref[...])
    out_ref[...] = jnp.dot(jax.nn.relu(hnew), linT_ref[...],
                           preferred_element_type=jnp.float32) + lb_ref[...]


def _row_spec():
    return pl.BlockSpec((_R, _OUT), lambda i: (i, 0))


def _parts_spec():
    return pl.BlockSpec((_NC, _R, _OUT), lambda i: (0, i, 0))


def _full_spec(r, c):
    return pl.BlockSpec((r, c), lambda i: (0, 0))


def _gru_mid(parts, h, wihT, whhT, bih, bhh, wn):
    return pl.pallas_call(
        _gru_mid_body,
        grid=(_N_NODES // _R,),
        in_specs=[_parts_spec(), _row_spec(),
                  _full_spec(_OUT, 3 * _OUT), _full_spec(_OUT, 3 * _OUT),
                  _full_spec(1, 3 * _OUT), _full_spec(1, 3 * _OUT),
                  _full_spec(_OUT, _OUT)],
        out_specs=[_row_spec(), _row_spec()],
        out_shape=[jax.ShapeDtypeStruct((_N_NODES, _OUT), jnp.float32),
                   jax.ShapeDtypeStruct((_N_NODES, _OUT), jnp.float32)],
    )(parts, h, wihT, whhT, bih, bhh, wn)


def _gru_last(parts, h, wihT, whhT, bih, bhh, linT, lb):
    return pl.pallas_call(
        _gru_last_body,
        grid=(_N_NODES // _R,),
        in_specs=[_parts_spec(), _row_spec(),
                  _full_spec(_OUT, 3 * _OUT), _full_spec(_OUT, 3 * _OUT),
                  _full_spec(1, 3 * _OUT), _full_spec(1, 3 * _OUT),
                  _full_spec(_OUT, _OUT), _full_spec(1, _OUT)],
        out_specs=_row_spec(),
        out_shape=jax.ShapeDtypeStruct((_N_NODES, _OUT), jnp.float32),
    )(parts, h, wihT, whhT, bih, bhh, linT, lb)


def kernel(x, edge_index, conv_weight, w_ih, w_hh, b_ih, b_hh, lin_w, lin_b):
    ei = edge_index.astype(jnp.int32)
    src = ei[0].reshape(_NW, _NCH, _CH)
    dst = ei[1].reshape(_NW, _NG, _DG, _CH)
    wihT = w_ih.T
    whhT = w_hh.T
    bih = b_ih.reshape(1, 3 * _OUT)
    bhh = b_hh.reshape(1, 3 * _OUT)
    linT = lin_w.T
    lb = lin_b.reshape(1, _OUT)

    h = x
    m = _matmul(x, conv_weight[0])
    out = None
    for i in range(_NUM_LAYERS):
        parts = _seg_sum(m, src, dst)
        if i < _NUM_LAYERS - 1:
            h, m = _gru_mid(parts, h, wihT, whhT, bih, bhh,
                            conv_weight[i + 1])
        else:
            out = _gru_last(parts, h, wihT, whhT, bih, bhh, linT, lb)
    return out


# speedup vs baseline: 1.0056x; 1.0056x over previous
import functools

import jax
import jax.numpy as jnp
from jax import lax
from jax.experimental import pallas as pl
from jax.experimental.pallas import tpu as pltpu
from jax.experimental.pallas import tpu_sc as plsc

_N_NODES = 10000
_N_EDGES = 320000
_OUT = 128
_NUM_LAYERS = 3

_NC, _NS = 2, 16
_NW = _NC * _NS
_EPW = _N_EDGES // _NW
_CH = 80
_NCH = _EPW // _CH
_DG = 25
_NG = _NCH // _DG
_NPAD = 10240
_RPS = _NPAD // _NS
_NWC = _RPS // _CH


def _seg_body(m_hbm, src_hbm, dst_hbm, out_hbm,
              srcs_v, dstg_v, rows_a, rows_b, acc_sh, sem_a, sem_b, sem_d):
    cid = lax.axis_index("c")
    sid = lax.axis_index("s")
    wid = cid * _NS + sid

    pltpu.make_async_copy(src_hbm.at[wid], srcs_v, sem_a).start()

    def zrow(i, carry):
        for j in range(_OUT // 16):
            rows_a[i, pl.ds(j * 16, 16)] = jnp.zeros((16,), jnp.float32)
        return carry
    lax.fori_loop(0, _CH, zrow, 0)

    def zcp(z, carry):
        pltpu.make_async_copy(
            rows_a, acc_sh.at[pl.ds(sid * _RPS + z * _CH, _CH)],
            sem_b).start()
        return carry
    lax.fori_loop(0, _NWC, zcp, 0)

    def zwait(z, carry):
        pltpu.make_async_copy(
            rows_a, acc_sh.at[pl.ds(sid * _RPS + z * _CH, _CH)],
            sem_b).wait()
        return carry
    lax.fori_loop(0, _NWC, zwait, 0)
    pltpu.make_async_copy(src_hbm.at[wid], srcs_v, sem_a).wait()
    pltpu.sync_copy(dst_hbm.at[wid, 0], dstg_v.at[pl.ds(0, _DG)])
    pltpu.make_async_copy(dst_hbm.at[wid, 1], dstg_v.at[pl.ds(_DG, _DG)],
                          sem_d).start()
    plsc.subcore_barrier()

    def load_dst_group(c):
        g = c // _DG

        @pl.when(jnp.logical_and(c % _DG == 0, g > 0))
        def _():
            pltpu.make_async_copy(
                dst_hbm.at[wid, g],
                dstg_v.at[pl.ds((g % 2) * _DG, _DG)], sem_d).wait()

            @pl.when(g + 1 < _NG)
            def _():
                pltpu.make_async_copy(
                    dst_hbm.at[wid, g + 1],
                    dstg_v.at[pl.ds(((g + 1) % 2) * _DG, _DG)],
                    sem_d).start()

    def gather_start(c, buf, sem):
        pltpu.make_async_copy(m_hbm.at[srcs_v.at[c]], buf, sem).start()

    def gather_wait(c, buf, sem):
        pltpu.make_async_copy(m_hbm.at[srcs_v.at[c]], buf, sem).wait()

    def scatter(c, buf):
        row = (c // _DG % 2) * _DG + c % _DG
        pltpu.sync_copy(buf, acc_sh.at[dstg_v.at[row]], add=True)

    gather_start(0, rows_a, sem_a)

    def pair(kk, carry):
        c0 = kk * 2
        gather_start(c0 + 1, rows_b, sem_b)
        load_dst_group(c0)
        gather_wait(c0, rows_a, sem_a)
        scatter(c0, rows_a)
        gather_start(c0 + 2, rows_a, sem_a)
        load_dst_group(c0 + 1)
        gather_wait(c0 + 1, rows_b, sem_b)
        scatter(c0 + 1, rows_b)
        return carry
    lax.fori_loop(0, (_NCH - 3) // 2, pair, 0)

    gather_start(_NCH - 2, rows_b, sem_b)
    gather_wait(_NCH - 3, rows_a, sem_a)
    scatter(_NCH - 3, rows_a)
    gather_start(_NCH - 1, rows_a, sem_a)
    load_dst_group(_NCH - 1)
    gather_wait(_NCH - 2, rows_b, sem_b)
    scatter(_NCH - 2, rows_b)
    gather_wait(_NCH - 1, rows_a, sem_a)
    scatter(_NCH - 1, rows_a)
    plsc.subcore_barrier()

    def wout(z, carry):
        r0 = sid * _RPS + z * _CH
        pltpu.sync_copy(acc_sh.at[pl.ds(r0, _CH)],
                        out_hbm.at[cid, pl.ds(r0, _CH)])
        return carry
    lax.fori_loop(0, _NWC, wout, 0)


_seg_sum = functools.partial(
    pl.kernel,
    out_type=jax.ShapeDtypeStruct((_NC, _NPAD, _OUT), jnp.float32),
    mesh=plsc.VectorSubcoreMesh(core_axis_name="c", subcore_axis_name="s"),
    scratch_types=[
        pltpu.VMEM((_NCH, _CH), jnp.int32),
        pltpu.VMEM((2 * _DG, _CH), jnp.int32),
        pltpu.VMEM((_CH, _OUT), jnp.float32),
        pltpu.VMEM((_CH, _OUT), jnp.float32),
        pltpu.VMEM_SHARED((_NPAD, _OUT), jnp.float32),
        pltpu.SemaphoreType.DMA,
        pltpu.SemaphoreType.DMA,
        pltpu.SemaphoreType.DMA,
    ],
)(_seg_body)


_R = 2000


def _mm_body(x_ref, w_ref, o_ref):
    o_ref[...] = jnp.dot(x_ref[...], w_ref[...],
                         preferred_element_type=jnp.float32)


def _matmul(x, w):
    return pl.pallas_call(
        _mm_body,
        grid=(_N_NODES // _R,),
        in_specs=[pl.BlockSpec((_R, _OUT), lambda i: (i, 0)),
                  pl.BlockSpec((_OUT, _OUT), lambda i: (0, 0))],
        out_specs=pl.BlockSpec((_R, _OUT), lambda i: (i, 0)),
        out_shape=jax.ShapeDtypeStruct((_N_NODES, _OUT), jnp.float32),
    )(x, w)


def _gru_math(parts_ref, h, wihT, whhT, bih, bhh):
    agg = parts_ref[0] + parts_ref[1]
    gi = jnp.dot(agg, wihT, preferred_element_type=jnp.float32) + bih
    gh = jnp.dot(h, whhT, preferred_element_type=jnp.float32) + bhh
    r = jax.nn.sigmoid(gi[:, :_OUT] + gh[:, :_OUT])
    z = jax.nn.sigmoid(gi[:, _OUT:2 * _OUT] + gh[:, _OUT:2 * _OUT])
    n = jnp.tanh(gi[:, 2 * _OUT:] + r * gh[:, 2 * _OUT:])
    return (1.0 - z) * n + z * h


def _gru_mid_body(parts_ref, h_ref, wihT_ref, whhT_ref, bih_ref,
                  bhh_ref, wn_ref, hnew_ref, mnext_ref):
    hnew = _gru_math(parts_ref, h_ref[...], wihT_ref[...],
                     whhT_ref[...], bih_ref[...], bhh_ref[...])
    hnew_ref[...] = hnew
    mnext_ref[...] = jnp.dot(hnew, wn_ref[...],
                             preferred_element_type=jnp.float32)


def _gru_last_body(parts_ref, h_ref, wihT_ref, whhT_ref, bih_ref,
                   bhh_ref, linT_ref, lb_ref, out_ref):
    hnew = _gru_math(parts_ref, h_ref[...], wihT_ref[...],
                     whhT_ref[...], bih_ref[...], bhh_ref[...])
    out_ref[...] = jnp.dot(jax.nn.relu(hnew), linT_ref[...],
                           preferred_element_type=jnp.float32) + lb_ref[...]


def _row_spec():
    return pl.BlockSpec((_R, _OUT), lambda i: (i, 0))


def _parts_spec():
    return pl.BlockSpec((_NC, _R, _OUT), lambda i: (0, i, 0))


def _full_spec(r, c):
    return pl.BlockSpec((r, c), lambda i: (0, 0))


def _gru_mid(parts, h, wihT, whhT, bih, bhh, wn):
    return pl.pallas_call(
        _gru_mid_body,
        grid=(_N_NODES // _R,),
        in_specs=[_parts_spec(), _row_spec(),
                  _full_spec(_OUT, 3 * _OUT), _full_spec(_OUT, 3 * _OUT),
                  _full_spec(1, 3 * _OUT), _full_spec(1, 3 * _OUT),
                  _full_spec(_OUT, _OUT)],
        out_specs=[_row_spec(), _row_spec()],
        out_shape=[jax.ShapeDtypeStruct((_N_NODES, _OUT), jnp.float32),
                   jax.ShapeDtypeStruct((_N_NODES, _OUT), jnp.float32)],
    )(parts, h, wihT, whhT, bih, bhh, wn)


def _gru_last(parts, h, wihT, whhT, bih, bhh, linT, lb):
    return pl.pallas_call(
        _gru_last_body,
        grid=(_N_NODES // _R,),
        in_specs=[_parts_spec(), _row_spec(),
                  _full_spec(_OUT, 3 * _OUT), _full_spec(_OUT, 3 * _OUT),
                  _full_spec(1, 3 * _OUT), _full_spec(1, 3 * _OUT),
                  _full_spec(_OUT, _OUT), _full_spec(1, _OUT)],
        out_specs=_row_spec(),
        out_shape=jax.ShapeDtypeStruct((_N_NODES, _OUT), jnp.float32),
    )(parts, h, wihT, whhT, bih, bhh, linT, lb)


def kernel(x, edge_index, conv_weight, w_ih, w_hh, b_ih, b_hh, lin_w, lin_b):
    ei = edge_index.astype(jnp.int32)
    src = ei[0].reshape(_NW, _NCH, _CH)
    dst = ei[1].reshape(_NW, _NG, _DG, _CH)
    wihT = w_ih.T
    whhT = w_hh.T
    bih = b_ih.reshape(1, 3 * _OUT)
    bhh = b_hh.reshape(1, 3 * _OUT)
    linT = lin_w.T
    lb = lin_b.reshape(1, _OUT)

    h = x
    m = _matmul(x, conv_weight[0])
    out = None
    for i in range(_NUM_LAYERS):
        parts = _seg_sum(m, src, dst)
        if i < _NUM_LAYERS - 1:
            h, m = _gru_mid(parts, h, wihT, whhT, bih, bhh,
                            conv_weight[i + 1])
        else:
            out = _gru_last(parts, h, wihT, whhT, bih, bhh, linT, lb)
    return out
